# baseline (speedup 1.0000x reference)
import functools

import jax
import jax.numpy as jnp
from jax import lax
from jax.experimental import pallas as pl
from jax.experimental.pallas import tpu as pltpu
from jax.experimental.pallas import tpu_sc as plsc

_NC, _NS, _L = 2, 16, 16
_NW = _NC * _NS
_N = 2048
_NVR = _N // _L
_CHUNK_ROWS = 16
_UF = 8
_N_ITERS = 10

_ZV = functools.partial(jnp.zeros, (_L,), jnp.float32)

_GATHER_DNUMS = lax.GatherDimensionNumbers(
    offset_dims=(), collapsed_slice_dims=(0,), start_index_map=(0,))


def _lane_gather(v, idx):
    return lax.gather(v, idx[:, None], _GATHER_DNUMS, slice_sizes=(1,),
                      mode=lax.GatherScatterMode.PROMISE_IN_BOUNDS)


def _lane_allreduce(v, op):
    idx = lax.iota(jnp.int32, _L)
    for shift in (1, 2, 4, 8):
        v = op(v, _lane_gather(v, lax.bitwise_xor(idx, shift)))
    return v


def _row_reduce(xv, row_off, fn):
    def body(j, accs):
        a0, a1, a2, a3 = accs
        off = row_off + j * (_L * _UF)
        a0 = a0 + fn(xv[pl.ds(off + 0 * _L, _L)])
        a1 = a1 + fn(xv[pl.ds(off + 1 * _L, _L)])
        a2 = a2 + fn(xv[pl.ds(off + 2 * _L, _L)])
        a3 = a3 + fn(xv[pl.ds(off + 3 * _L, _L)])
        a0 = a0 + fn(xv[pl.ds(off + 4 * _L, _L)])
        a1 = a1 + fn(xv[pl.ds(off + 5 * _L, _L)])
        a2 = a2 + fn(xv[pl.ds(off + 6 * _L, _L)])
        a3 = a3 + fn(xv[pl.ds(off + 7 * _L, _L)])
        return a0, a1, a2, a3

    a0, a1, a2, a3 = lax.fori_loop(0, _NVR // _UF, body,
                                   (_ZV(), _ZV(), _ZV(), _ZV()))
    return (a0 + a1) + (a2 + a3)


def _sc_body(x_hbm, o_hbm, xv, ov):
    wid = lax.axis_index("s") * _NC + lax.axis_index("c")
    rows_total = x_hbm.shape[0] // _N
    rpw = rows_total // _NW
    nchunks = rpw // _CHUNK_ROWS
    base = wid * rpw

    def chunk_body(c, _c):
        row0 = (base + c * _CHUNK_ROWS) * _N
        pltpu.sync_copy(x_hbm.at[pl.ds(row0, _CHUNK_ROWS * _N)], xv)

        def row_body(r, _r):
            off = r * _N

            def max_b(j, acc):
                return jnp.maximum(acc, xv[pl.ds(off + j * _L, _L)])
            acc = lax.fori_loop(1, _NVR, max_b, xv[pl.ds(off, _L)])
            mx = _lane_allreduce(acc, jnp.maximum)
            lo = mx - 1.0
            hi = mx

            def pass_b(t, lohi):
                lo, hi = lohi
                mid = 0.5 * (lo + hi)
                a = _row_reduce(xv, off,
                                lambda v: jnp.maximum(v - mid, 0.0))
                f = _lane_allreduce(a, jnp.add)
                gt = f > 1.0
                return jnp.where(gt, mid, lo), jnp.where(gt, hi, mid)

            lo, hi = lax.fori_loop(0, _N_ITERS, pass_b, (lo, hi))

            s = _row_reduce(xv, off, lambda v: jnp.where(v > lo, v, 0.0))
            k = _row_reduce(xv, off, lambda v: jnp.where(v > lo, 1.0, 0.0))
            ss = _lane_allreduce(s, jnp.add)
            kk = _lane_allreduce(k, jnp.add)
            tau = jnp.clip((ss - 1.0) / kk, lo, hi)

            def out_b(j, _o):
                o = off + j * _L
                ov[pl.ds(o, _L)] = jnp.maximum(xv[pl.ds(o, _L)] - tau, 0.0)
                return 0
            lax.fori_loop(0, _NVR, out_b, 0)
            return 0

        lax.fori_loop(0, _CHUNK_ROWS, row_body, 0)
        pltpu.sync_copy(ov, o_hbm.at[pl.ds(row0, _CHUNK_ROWS * _N)])
        return 0

    lax.fori_loop(0, nchunks, chunk_body, 0)


def kernel(input):
    orig_shape = input.shape
    n = orig_shape[-1]
    x1 = input.reshape(-1)
    rows = x1.shape[0] // n
    mesh = plsc.VectorSubcoreMesh(core_axis_name="c", subcore_axis_name="s")
    out = pl.kernel(
        _sc_body,
        mesh=mesh,
        out_type=jax.ShapeDtypeStruct((rows * n,), jnp.float32),
        scratch_types=[
            pltpu.VMEM((_CHUNK_ROWS * _N,), jnp.float32),
            pltpu.VMEM((_CHUNK_ROWS * _N,), jnp.float32),
        ],
    )(x1)
    return out.reshape(orig_shape)


# speedup vs baseline: 8.7176x; 8.7176x over previous
import jax
import jax.numpy as jnp
from jax.experimental import pallas as pl

_N_ITERS_FAST = 6
_N_ITERS_EXACT = 3
_BLOCK_ROWS = 1024


def _sparsemax_block(x_ref, o_ref):
    x = x_ref[...]
    n = x.shape[1]
    mx = jnp.max(x, axis=1, keepdims=True)
    lo = mx - 1.0
    hi = mx

    def body_fast(_, carry):
        lo, hi = carry
        mid = 0.5 * (lo + hi)
        sm = jnp.sum(jnp.maximum(x, mid), axis=1, keepdims=True)
        gt = sm > 1.0 + n * mid
        lo = jnp.where(gt, mid, lo)
        hi = jnp.where(gt, hi, mid)
        return lo, hi

    def body_exact(_, carry):
        lo, hi = carry
        mid = 0.5 * (lo + hi)
        f = jnp.sum(jnp.maximum(x - mid, 0.0), axis=1, keepdims=True)
        gt = f > 1.0
        lo = jnp.where(gt, mid, lo)
        hi = jnp.where(gt, hi, mid)
        return lo, hi

    lo, hi = jax.lax.fori_loop(0, _N_ITERS_FAST, body_fast, (lo, hi))
    lo, hi = jax.lax.fori_loop(0, _N_ITERS_EXACT, body_exact, (lo, hi))
    mask = x > lo
    s = jnp.sum(jnp.where(mask, x, 0.0), axis=1, keepdims=True)
    k = jnp.sum(mask.astype(x.dtype), axis=1, keepdims=True)
    tau = jnp.clip((s - 1.0) / k, lo, hi)
    o_ref[...] = jnp.maximum(x - tau, 0.0)


def kernel(input):
    orig_shape = input.shape
    n = orig_shape[-1]
    x2 = input.reshape(-1, n)
    rows = x2.shape[0]
    out = pl.pallas_call(
        _sparsemax_block,
        grid=(rows // _BLOCK_ROWS,),
        in_specs=[pl.BlockSpec((_BLOCK_ROWS, n), lambda i: (i, 0))],
        out_specs=pl.BlockSpec((_BLOCK_ROWS, n), lambda i: (i, 0)),
        out_shape=jax.ShapeDtypeStruct((rows, n), x2.dtype),
    )(x2)
    return out.reshape(orig_shape)


# speedup vs baseline: 10.0746x; 1.1557x over previous
import jax
import jax.numpy as jnp
from jax.experimental import pallas as pl

_N_ITERS_FAST = 6
_N_ITERS_EXACT = 0
_BLOCK_ROWS = 1024


def _sparsemax_block(x_ref, o_ref):
    x = x_ref[...]
    n = x.shape[1]
    mx = jnp.max(x, axis=1, keepdims=True)
    lo = mx - 1.0
    hi = mx

    def body_fast(_, carry):
        lo, hi = carry
        mid = 0.5 * (lo + hi)
        sm = jnp.sum(jnp.maximum(x, mid), axis=1, keepdims=True)
        gt = sm > 1.0 + n * mid
        lo = jnp.where(gt, mid, lo)
        hi = jnp.where(gt, hi, mid)
        return lo, hi

    def body_exact(_, carry):
        lo, hi = carry
        mid = 0.5 * (lo + hi)
        f = jnp.sum(jnp.maximum(x - mid, 0.0), axis=1, keepdims=True)
        gt = f > 1.0
        lo = jnp.where(gt, mid, lo)
        hi = jnp.where(gt, hi, mid)
        return lo, hi

    lo, hi = jax.lax.fori_loop(0, _N_ITERS_FAST, body_fast, (lo, hi))
    lo, hi = jax.lax.fori_loop(0, _N_ITERS_EXACT, body_exact, (lo, hi))

    def newton(t):
        mask = x > t
        s = jnp.sum(jnp.where(mask, x, 0.0), axis=1, keepdims=True)
        k = jnp.sum(mask.astype(x.dtype), axis=1, keepdims=True)
        return jnp.clip((s - 1.0) / k, t, hi)

    tau = newton(newton(lo))
    o_ref[...] = jnp.maximum(x - tau, 0.0)


def kernel(input):
    orig_shape = input.shape
    n = orig_shape[-1]
    x2 = input.reshape(-1, n)
    rows = x2.shape[0]
    out = pl.pallas_call(
        _sparsemax_block,
        grid=(rows // _BLOCK_ROWS,),
        in_specs=[pl.BlockSpec((_BLOCK_ROWS, n), lambda i: (i, 0))],
        out_specs=pl.BlockSpec((_BLOCK_ROWS, n), lambda i: (i, 0)),
        out_shape=jax.ShapeDtypeStruct((rows, n), x2.dtype),
    )(x2)
    return out.reshape(orig_shape)


# speedup vs baseline: 10.9364x; 1.0855x over previous
import jax
import jax.numpy as jnp
from jax.experimental import pallas as pl

_N_ITERS_FAST = 5
_N_ITERS_EXACT = 0
_BLOCK_ROWS = 1024


def _sparsemax_block(x_ref, o_ref):
    x = x_ref[...]
    n = x.shape[1]
    mx = jnp.max(x, axis=1, keepdims=True)
    lo = mx - 1.0
    hi = mx

    def body_fast(_, carry):
        lo, hi = carry
        mid = 0.5 * (lo + hi)
        sm = jnp.sum(jnp.maximum(x, mid), axis=1, keepdims=True)
        gt = sm > 1.0 + n * mid
        lo = jnp.where(gt, mid, lo)
        hi = jnp.where(gt, hi, mid)
        return lo, hi

    def body_exact(_, carry):
        lo, hi = carry
        mid = 0.5 * (lo + hi)
        f = jnp.sum(jnp.maximum(x - mid, 0.0), axis=1, keepdims=True)
        gt = f > 1.0
        lo = jnp.where(gt, mid, lo)
        hi = jnp.where(gt, hi, mid)
        return lo, hi

    lo, hi = jax.lax.fori_loop(0, _N_ITERS_FAST, body_fast, (lo, hi))
    lo, hi = jax.lax.fori_loop(0, _N_ITERS_EXACT, body_exact, (lo, hi))

    def newton(t):
        mask = x > t
        s = jnp.sum(jnp.where(mask, x, 0.0), axis=1, keepdims=True)
        k = jnp.sum(mask.astype(x.dtype), axis=1, keepdims=True)
        return jnp.clip((s - 1.0) / k, t, hi)

    tau = newton(newton(lo))
    o_ref[...] = jnp.maximum(x - tau, 0.0)


def kernel(input):
    orig_shape = input.shape
    n = orig_shape[-1]
    x2 = input.reshape(-1, n)
    rows = x2.shape[0]
    out = pl.pallas_call(
        _sparsemax_block,
        grid=(rows // _BLOCK_ROWS,),
        in_specs=[pl.BlockSpec((_BLOCK_ROWS, n), lambda i: (i, 0))],
        out_specs=pl.BlockSpec((_BLOCK_ROWS, n), lambda i: (i, 0)),
        out_shape=jax.ShapeDtypeStruct((rows, n), x2.dtype),
    )(x2)
    return out.reshape(orig_shape)


# speedup vs baseline: 11.0308x; 1.0086x over previous
import jax
import jax.numpy as jnp
from jax.experimental import pallas as pl

_N_ITERS_FAST = 5
_N_ITERS_EXACT = 0
_BLOCK_ROWS = 512


def _sparsemax_block(x_ref, o_ref):
    x = x_ref[...]
    n = x.shape[1]
    mx = jnp.max(x, axis=1, keepdims=True)
    lo = mx - 1.0
    hi = mx

    def body_fast(_, carry):
        lo, hi = carry
        mid = 0.5 * (lo + hi)
        sm = jnp.sum(jnp.maximum(x, mid), axis=1, keepdims=True)
        gt = sm > 1.0 + n * mid
        lo = jnp.where(gt, mid, lo)
        hi = jnp.where(gt, hi, mid)
        return lo, hi

    def body_exact(_, carry):
        lo, hi = carry
        mid = 0.5 * (lo + hi)
        f = jnp.sum(jnp.maximum(x - mid, 0.0), axis=1, keepdims=True)
        gt = f > 1.0
        lo = jnp.where(gt, mid, lo)
        hi = jnp.where(gt, hi, mid)
        return lo, hi

    lo, hi = jax.lax.fori_loop(0, _N_ITERS_FAST, body_fast, (lo, hi))
    lo, hi = jax.lax.fori_loop(0, _N_ITERS_EXACT, body_exact, (lo, hi))

    def newton(t):
        mask = x > t
        s = jnp.sum(jnp.where(mask, x, 0.0), axis=1, keepdims=True)
        k = jnp.sum(mask.astype(x.dtype), axis=1, keepdims=True)
        return jnp.clip((s - 1.0) / k, t, hi)

    tau = newton(newton(lo))
    o_ref[...] = jnp.maximum(x - tau, 0.0)


def kernel(input):
    orig_shape = input.shape
    n = orig_shape[-1]
    x2 = input.reshape(-1, n)
    rows = x2.shape[0]
    out = pl.pallas_call(
        _sparsemax_block,
        grid=(rows // _BLOCK_ROWS,),
        in_specs=[pl.BlockSpec((_BLOCK_ROWS, n), lambda i: (i, 0))],
        out_specs=pl.BlockSpec((_BLOCK_ROWS, n), lambda i: (i, 0)),
        out_shape=jax.ShapeDtypeStruct((rows, n), x2.dtype),
    )(x2)
    return out.reshape(orig_shape)
